# baseline (speedup 1.0000x reference)
import jax
import jax.numpy as jnp
from jax import lax
from jax.experimental import pallas as pl
from jax.experimental.pallas import tpu as pltpu
from jax.experimental.pallas import tpu_sc as plsc

_B = 4096
_N = 128
_T = 16
_TOPK = 6
_DIST_THRESH = 120.0
_NW = 32
_RW = _B // _NW
_L = 16
_NG = _RW // _L
_SENT = 2 ** 30


def _sc_body(big_hbm, outs_hbm, outi_hbm, big_v, outs_v, outi_v):
    nc = lax.axis_index("c")
    ns = lax.axis_index("s")
    wid = ns * 2 + nc

    pltpu.sync_copy(big_hbm.at[wid], big_v)

    neg_inf = jnp.float32(-jnp.inf)

    def group(g, _):
        lanes = pl.ds(g * _L, _L)
        ex = big_v[6 * _N + 0, lanes]
        ey = big_v[6 * _N + 1, lanes]
        ev = big_v[6 * _N + 2, lanes]
        el = big_v[6 * _N + 3, lanes]

        def p1(k, hc):
            for u in range(4):
                n = k * 4 + u
                occ = big_v[5 * _N + n, lanes]
                dist = big_v[4 * _N + n, lanes]
                hc = jnp.where((occ > 0.5) & (dist <= _DIST_THRESH),
                               jnp.float32(1.0), hc)
            return hc

        hc = lax.fori_loop(0, _N // 4, p1, jnp.zeros((_L,), jnp.float32))
        no_close = hc < 0.5

        def scoremask(n):
            x = big_v[n, lanes]
            y = big_v[_N + n, lanes]
            vv = big_v[2 * _N + n, lanes]
            lane = big_v[3 * _N + n, lanes]
            dist = big_v[4 * _N + n, lanes]
            occ = big_v[5 * _N + n, lanes]

            ld = jnp.abs(lane - el)
            bonus = jnp.where(ld < 0.5, jnp.float32(0.2), jnp.float32(0.0))
            bonus = bonus + jnp.where(jnp.abs(ld - 1.0) < 0.5,
                                      jnp.float32(0.1), jnp.float32(0.0))
            dx = jnp.abs(x - ex)
            dy = jnp.abs(y - ey)
            closing = jnp.maximum(ev - vv, 0.0)
            score = (1.25 / (dy + 1.0) + 0.75 / (dist + 1.0)
                     + 0.25 * jnp.minimum(closing * 0.1, 2.0)
                     + bonus + 0.15 / (dx + 1.0))
            avail = (occ > 0.5) & ((dist <= _DIST_THRESH) | no_close)
            return jnp.where(avail, score, neg_inf)

        minf = jnp.full((_L,), neg_inf)
        sent = jnp.full((_L,), _SENT, jnp.int32)
        s = [minf] * _TOPK
        i = [sent] * _TOPK
        for n in range(_TOPK):
            v = scoremask(n)
            n_vec = jnp.full((_L,), n, jnp.int32)
            c = [(v > s[j]) | ((v == s[j]) & (n_vec < i[j]))
                 for j in range(_TOPK)]
            ns_ = [jnp.where(c[0], v, s[0])]
            ni_ = [jnp.where(c[0], n_vec, i[0])]
            for j in range(1, _TOPK):
                ns_.append(jnp.where(c[j], jnp.where(c[j - 1], s[j - 1], v),
                                     s[j]))
                ni_.append(jnp.where(c[j], jnp.where(c[j - 1], i[j - 1],
                                                     n_vec), i[j]))
            s, i = ns_, ni_

        def ins_strict(n, carry):
            s0, s1, s2, s3, s4, s5, i0, i1, i2, i3, i4, i5 = carry
            v = scoremask(n)
            n_vec = jnp.zeros((_L,), jnp.int32) + n
            c0 = v > s0
            c1 = v > s1
            c2 = v > s2
            c3 = v > s3
            c4 = v > s4
            c5 = v > s5
            ns0 = jnp.where(c0, v, s0)
            ni0 = jnp.where(c0, n_vec, i0)
            ns1 = jnp.where(c1, jnp.where(c0, s0, v), s1)
            ni1 = jnp.where(c1, jnp.where(c0, i0, n_vec), i1)
            ns2 = jnp.where(c2, jnp.where(c1, s1, v), s2)
            ni2 = jnp.where(c2, jnp.where(c1, i1, n_vec), i2)
            ns3 = jnp.where(c3, jnp.where(c2, s2, v), s3)
            ni3 = jnp.where(c3, jnp.where(c2, i2, n_vec), i3)
            ns4 = jnp.where(c4, jnp.where(c3, s3, v), s4)
            ni4 = jnp.where(c4, jnp.where(c3, i3, n_vec), i4)
            ns5 = jnp.where(c5, jnp.where(c4, s4, v), s5)
            ni5 = jnp.where(c5, jnp.where(c4, i4, n_vec), i5)
            return (ns0, ns1, ns2, ns3, ns4, ns5,
                    ni0, ni1, ni2, ni3, ni4, ni5)

        def p2(k, carry):
            carry = ins_strict(_TOPK + 2 * k, carry)
            return ins_strict(_TOPK + 2 * k + 1, carry)

        res = lax.fori_loop(0, (_N - _TOPK) // 2, p2, tuple(s) + tuple(i))
        for j in range(_TOPK):
            outs_v[j, lanes] = res[j]
            outi_v[j, lanes] = res[_TOPK + j]
        return 0

    lax.fori_loop(0, _NG, group, 0)

    pltpu.sync_copy(outs_v, outs_hbm.at[wid])
    pltpu.sync_copy(outi_v, outi_hbm.at[wid])


@jax.jit
def _sc_call(big):
    mesh = plsc.VectorSubcoreMesh(core_axis_name="c", subcore_axis_name="s")
    return pl.kernel(
        _sc_body,
        out_type=[jax.ShapeDtypeStruct((_NW, _TOPK, _RW), jnp.float32),
                  jax.ShapeDtypeStruct((_NW, _TOPK, _RW), jnp.int32)],
        mesh=mesh,
        compiler_params=pltpu.CompilerParams(needs_layout_passes=False),
        scratch_types=[
            pltpu.VMEM((6 * _N + 4, _RW), jnp.float32),
            pltpu.VMEM((_TOPK, _RW), jnp.float32),
            pltpu.VMEM((_TOPK, _RW), jnp.int32),
        ],
    )(big)


def kernel(social_occ, ego_state_raw, nbr_state_raw_grid, ego_lane,
           nbr_lane_grid, nbr_dist_grid):
    nbr_last = nbr_state_raw_grid[:, :, -1, :]

    def slab(p):
        return p.reshape(_NW, _RW, _N).transpose(0, 2, 1)

    ego_slab = jnp.concatenate(
        [ego_state_raw[:, -1, :3], ego_lane[:, -1, :]],
        axis=1).reshape(_NW, _RW, 4).transpose(0, 2, 1)
    big = jnp.concatenate(
        [slab(nbr_last[:, :, 0]), slab(nbr_last[:, :, 1]),
         slab(nbr_last[:, :, 2]), slab(nbr_lane_grid[:, :, -1, 0]),
         slab(nbr_dist_grid[:, :, -1, 0]),
         slab(social_occ.astype(jnp.float32)), ego_slab], axis=1)

    outs, outi = _sc_call(big)
    topk_score = outs.transpose(0, 2, 1).reshape(_B, _TOPK)
    selected_idx = outi.transpose(0, 2, 1).reshape(_B, _TOPK)
    selected_valid = jnp.isfinite(topk_score)
    return topk_score, selected_idx, selected_valid


# speedup vs baseline: 1.1808x; 1.1808x over previous
import jax
import jax.numpy as jnp
from jax import lax
from jax.experimental import pallas as pl
from jax.experimental.pallas import tpu as pltpu
from jax.experimental.pallas import tpu_sc as plsc

_B = 4096
_N = 128
_T = 16
_TOPK = 6
_DIST_THRESH = 120.0
_NW = 32
_RW = _B // _NW
_L = 16
_NG = _RW // _L
_SENT = 2 ** 30


def _sc_body(x_hbm, y_hbm, v_hbm, lane_hbm, dist_hbm, occ_hbm, ego_hbm,
             outs_hbm, outi_hbm,
             x_v, y_v, v_v, lane_v, dist_v, occ_v, ego_v, outs_v, outi_v):
    nc = lax.axis_index("c")
    ns = lax.axis_index("s")
    wid = ns * 2 + nc
    base = wid * _RW

    cols = pl.ds(base, _RW)
    pltpu.sync_copy(x_hbm.at[:, cols], x_v)
    pltpu.sync_copy(y_hbm.at[:, cols], y_v)
    pltpu.sync_copy(v_hbm.at[:, cols], v_v)
    pltpu.sync_copy(lane_hbm.at[:, cols], lane_v)
    pltpu.sync_copy(dist_hbm.at[:, cols], dist_v)
    pltpu.sync_copy(occ_hbm.at[:, cols], occ_v)
    pltpu.sync_copy(ego_hbm.at[:, cols], ego_v)

    neg_inf = jnp.float32(-jnp.inf)

    def group(g, _):
        lanes = pl.ds(g * _L, _L)
        ex = ego_v[0, lanes]
        ey = ego_v[1, lanes]
        ev = ego_v[2, lanes]
        el = ego_v[3, lanes]

        def p1(k, hc):
            for u in range(4):
                n = k * 4 + u
                occ = occ_v[n, lanes]
                dist = dist_v[n, lanes]
                hc = jnp.where((occ > 0.5) & (dist <= _DIST_THRESH),
                               jnp.float32(1.0), hc)
            return hc

        hc = lax.fori_loop(0, _N // 4, p1, jnp.zeros((_L,), jnp.float32))
        no_close = hc < 0.5

        def scoremask(n):
            x = x_v[n, lanes]
            y = y_v[n, lanes]
            vv = v_v[n, lanes]
            lane = lane_v[n, lanes]
            dist = dist_v[n, lanes]
            occ = occ_v[n, lanes]

            ld = jnp.abs(lane - el)
            bonus = jnp.where(ld < 0.5, jnp.float32(0.2), jnp.float32(0.0))
            bonus = bonus + jnp.where(jnp.abs(ld - 1.0) < 0.5,
                                      jnp.float32(0.1), jnp.float32(0.0))
            dx = jnp.abs(x - ex)
            dy = jnp.abs(y - ey)
            closing = jnp.maximum(ev - vv, 0.0)
            score = (1.25 / (dy + 1.0) + 0.75 / (dist + 1.0)
                     + 0.25 * jnp.minimum(closing * 0.1, 2.0)
                     + bonus + 0.15 / (dx + 1.0))
            avail = (occ > 0.5) & ((dist <= _DIST_THRESH) | no_close)
            return jnp.where(avail, score, neg_inf)

        minf = jnp.full((_L,), neg_inf)
        sent = jnp.full((_L,), _SENT, jnp.int32)
        s = [minf] * _TOPK
        i = [sent] * _TOPK
        for n in range(_TOPK):
            v = scoremask(n)
            n_vec = jnp.full((_L,), n, jnp.int32)
            c = [(v > s[j]) | ((v == s[j]) & (n_vec < i[j]))
                 for j in range(_TOPK)]
            ns_ = [jnp.where(c[0], v, s[0])]
            ni_ = [jnp.where(c[0], n_vec, i[0])]
            for j in range(1, _TOPK):
                ns_.append(jnp.where(c[j], jnp.where(c[j - 1], s[j - 1], v),
                                     s[j]))
                ni_.append(jnp.where(c[j], jnp.where(c[j - 1], i[j - 1],
                                                     n_vec), i[j]))
            s, i = ns_, ni_

        def ins_strict(n, carry):
            s0, s1, s2, s3, s4, s5, i0, i1, i2, i3, i4, i5 = carry
            v = scoremask(n)
            n_vec = jnp.zeros((_L,), jnp.int32) + n
            c0 = v > s0
            c1 = v > s1
            c2 = v > s2
            c3 = v > s3
            c4 = v > s4
            c5 = v > s5
            ns0 = jnp.where(c0, v, s0)
            ni0 = jnp.where(c0, n_vec, i0)
            ns1 = jnp.where(c1, jnp.where(c0, s0, v), s1)
            ni1 = jnp.where(c1, jnp.where(c0, i0, n_vec), i1)
            ns2 = jnp.where(c2, jnp.where(c1, s1, v), s2)
            ni2 = jnp.where(c2, jnp.where(c1, i1, n_vec), i2)
            ns3 = jnp.where(c3, jnp.where(c2, s2, v), s3)
            ni3 = jnp.where(c3, jnp.where(c2, i2, n_vec), i3)
            ns4 = jnp.where(c4, jnp.where(c3, s3, v), s4)
            ni4 = jnp.where(c4, jnp.where(c3, i3, n_vec), i4)
            ns5 = jnp.where(c5, jnp.where(c4, s4, v), s5)
            ni5 = jnp.where(c5, jnp.where(c4, i4, n_vec), i5)
            return (ns0, ns1, ns2, ns3, ns4, ns5,
                    ni0, ni1, ni2, ni3, ni4, ni5)

        def p2(k, carry):
            carry = ins_strict(_TOPK + 2 * k, carry)
            return ins_strict(_TOPK + 2 * k + 1, carry)

        res = lax.fori_loop(0, (_N - _TOPK) // 2, p2, tuple(s) + tuple(i))
        for j in range(_TOPK):
            outs_v[j, lanes] = res[j]
            outi_v[j, lanes] = res[_TOPK + j]
        return 0

    lax.fori_loop(0, _NG, group, 0)

    pltpu.sync_copy(outs_v, outs_hbm.at[wid])
    pltpu.sync_copy(outi_v, outi_hbm.at[wid])


@jax.jit
def _sc_call(x_t, y_t, v_t, lane_t, dist_t, occ_t, ego_t):
    mesh = plsc.VectorSubcoreMesh(core_axis_name="c", subcore_axis_name="s")
    return pl.kernel(
        _sc_body,
        out_type=[jax.ShapeDtypeStruct((_NW, _TOPK, _RW), jnp.float32),
                  jax.ShapeDtypeStruct((_NW, _TOPK, _RW), jnp.int32)],
        mesh=mesh,
        compiler_params=pltpu.CompilerParams(needs_layout_passes=False),
        scratch_types=[
            pltpu.VMEM((_N, _RW), jnp.float32),
            pltpu.VMEM((_N, _RW), jnp.float32),
            pltpu.VMEM((_N, _RW), jnp.float32),
            pltpu.VMEM((_N, _RW), jnp.float32),
            pltpu.VMEM((_N, _RW), jnp.float32),
            pltpu.VMEM((_N, _RW), jnp.float32),
            pltpu.VMEM((4, _RW), jnp.float32),
            pltpu.VMEM((_TOPK, _RW), jnp.float32),
            pltpu.VMEM((_TOPK, _RW), jnp.int32),
        ],
    )(x_t, y_t, v_t, lane_t, dist_t, occ_t, ego_t)


def kernel(social_occ, ego_state_raw, nbr_state_raw_grid, ego_lane,
           nbr_lane_grid, nbr_dist_grid):
    nbr_last = nbr_state_raw_grid[:, :, -1, :]
    x_t = nbr_last[:, :, 0].T
    y_t = nbr_last[:, :, 1].T
    v_t = nbr_last[:, :, 2].T
    lane_t = nbr_lane_grid[:, :, -1, 0].T
    dist_t = nbr_dist_grid[:, :, -1, 0].T
    occ_t = social_occ.T.astype(jnp.float32)
    ego_t = jnp.concatenate(
        [ego_state_raw[:, -1, :3], ego_lane[:, -1, :]], axis=1).T

    outs, outi = _sc_call(x_t, y_t, v_t, lane_t, dist_t, occ_t, ego_t)
    topk_score = outs.transpose(0, 2, 1).reshape(_B, _TOPK)
    selected_idx = outi.transpose(0, 2, 1).reshape(_B, _TOPK)
    selected_valid = jnp.isfinite(topk_score)
    return topk_score, selected_idx, selected_valid
